# baseline (speedup 1.0000x reference)
import math

import jax
import jax.numpy as jnp
from jax.experimental import pallas as pl
from jax.experimental.pallas import tpu as pltpu

N = 2048
I = 16
H = 16
O = 16
CUTS = 8
RADIUS = 9.0
SCALE = 1.0 / (math.sqrt(2.0) * RADIUS)
LOG2E = 1.4426950408889634
SQL2E = math.sqrt(LOG2E)

S = 32
NBLK = N // S
JC = 512
NCH = N // JC


def _lrelu(x):
    return jnp.where(x >= 0, x, 0.2 * x)


def _group_norm(x, gamma, beta, groups=4, eps=1e-5):
    c = x.shape[1]
    gs = c // groups
    cols = []
    for g in range(groups):
        sub = x[:, g * gs:(g + 1) * gs]
        m = jnp.mean(sub)
        v = jnp.mean((sub - m) * (sub - m))
        cols.append((sub - m) / jnp.sqrt(v + eps))
    return jnp.concatenate(cols, axis=1) * gamma + beta


def _pre_body(feat_ref, w1_ref, b1_ref, w2_ref, b2_ref, g_ref, be_ref, out_ref):
    x = _lrelu(jnp.dot(feat_ref[...], w1_ref[...],
                       preferred_element_type=jnp.float32) + b1_ref[...])
    x = _lrelu(jnp.dot(x, w2_ref[...],
                       preferred_element_type=jnp.float32) + b2_ref[...])
    out_ref[...] = _group_norm(x, g_ref[...], be_ref[...])


def _post_body(agg_ref, w1_ref, b1_ref, w2_ref, b2_ref, g_ref, be_ref, out_ref):
    x = _lrelu(jnp.dot(agg_ref[...], w1_ref[...],
                       preferred_element_type=jnp.float32) + b1_ref[...])
    x = _lrelu(jnp.dot(x, w2_ref[...],
                       preferred_element_type=jnp.float32) + b2_ref[...])
    out_ref[...] = _group_norm(x, g_ref[...], be_ref[...])


def _main_body(r_ref, nuv9_ref, pblk_ref, m_ref, bc1_ref, wbig_ref,
               bc2_ref, ft_ref, out_ref, xst_ref):
    pb = pblk_ref[...] * SCALE
    qi = jnp.sum(pb * pb, axis=1, keepdims=True)
    nuv9 = nuv9_ref[...]
    ni = nuv9[:, 0:3]

    a24 = jnp.dot(nuv9, m_ref[...], preferred_element_type=jnp.float32)

    z3 = jnp.zeros((S, 3), jnp.float32)
    z1 = jnp.zeros((S, 1), jnp.float32)
    o1 = jnp.ones((S, 1), jnp.float32)
    planes = [
        jnp.concatenate([ni, z3, z1, z1], axis=1),
        jnp.concatenate([z3, (2.0 * LOG2E) * pb, -LOG2E * o1, -LOG2E * qi],
                        axis=1),
    ]
    for c in range(CUTS):
        ac = jnp.concatenate(
            [a24[:, c:c + 1], a24[:, 8 + c:9 + c], a24[:, 16 + c:17 + c]],
            axis=1)
        cc = bc1_ref[0, c] - jnp.sum(ac * pb, axis=1, keepdims=True)
        planes.append(jnp.concatenate([z3, ac, z1, cc], axis=1))
    lmat = jnp.concatenate(planes, axis=0)

    u = jnp.dot(lmat, r_ref[...], preferred_element_type=jnp.float32,
                precision=jax.lax.Precision.HIGHEST)
    dot = u[0:S, :]
    negd = u[S:2 * S, :]
    t = 2.0 - dot
    window = jnp.exp2(negd * t * t)

    for c in range(CUTS):
        xst_ref[c * S:(c + 1) * S, :] = jnp.maximum(
            u[(2 + c) * S:(3 + c) * S, :], 0.0).astype(jnp.bfloat16)

    xst = xst_ref[...]
    wbig = wbig_ref[...]
    y0 = jnp.dot(wbig[:H * S // 2, :], xst,
                 preferred_element_type=jnp.float32)
    y1 = jnp.dot(wbig[H * S // 2:, :], xst,
                 preferred_element_type=jnp.float32)
    cols = []
    for h in range(H):
        yh = y0 if h < 8 else y1
        hh = h if h < 8 else h - 8
        xh = jnp.maximum(yh[hh * S:(hh + 1) * S, :] + bc2_ref[0, h], 0.0)
        th = window * xh * ft_ref[h:h + 1, :]
        cols.append(jnp.sum(th, axis=1, keepdims=True))
    out_ref[...] = jnp.concatenate(cols, axis=1)


def kernel(points, nuv, features, W_in1, b_in1, W_in2, b_in2, g_in, be_in,
           Wc1, bc1, Wc2, bc2, W_out1, b_out1, W_out2, b_out2, g_out, be_out):
    nuv9 = nuv.reshape(N, 9)
    pts = points.T * SCALE
    qj = jnp.sum(pts * pts, axis=0, keepdims=True)
    nt = nuv[:, 0, :].T
    rmat = jnp.concatenate(
        [nt, pts, qj, jnp.ones((1, N), jnp.float32)], axis=0)
    m = (Wc1.T[:, None, None, :]
         * jnp.eye(3, dtype=jnp.float32)[None, :, :, None]).reshape(9, 24)
    wbig = (Wc2[:, None, :, None]
            * jnp.eye(S, dtype=jnp.float32)[None, :, None, :]).reshape(
                H * S, CUTS * S).astype(jnp.bfloat16)

    f = pl.pallas_call(
        _pre_body,
        out_shape=jax.ShapeDtypeStruct((N, H), jnp.float32),
    )(features, W_in1.T, b_in1.reshape(1, -1), W_in2.T, b_in2.reshape(1, -1),
      g_in.reshape(1, -1), be_in.reshape(1, -1))
    ft = f.T

    agg = pl.pallas_call(
        _main_body,
        grid=(NBLK,),
        in_specs=[
            pl.BlockSpec((8, N), lambda b: (0, 0)),
            pl.BlockSpec((S, 9), lambda b: (b, 0)),
            pl.BlockSpec((S, 3), lambda b: (b, 0)),
            pl.BlockSpec((9, 24), lambda b: (0, 0)),
            pl.BlockSpec((1, CUTS), lambda b: (0, 0)),
            pl.BlockSpec((H * S, CUTS * S), lambda b: (0, 0)),
            pl.BlockSpec((1, H), lambda b: (0, 0)),
            pl.BlockSpec((H, N), lambda b: (0, 0)),
        ],
        out_specs=pl.BlockSpec((S, H), lambda b: (b, 0)),
        out_shape=jax.ShapeDtypeStruct((N, H), jnp.float32),
        scratch_shapes=[pltpu.VMEM((CUTS * S, N), jnp.bfloat16)],
    )(rmat, nuv9, points, m, bc1.reshape(1, -1), wbig, bc2.reshape(1, -1),
      ft)

    out = pl.pallas_call(
        _post_body,
        out_shape=jax.ShapeDtypeStruct((N, O), jnp.float32),
    )(agg, W_out1.T, b_out1.reshape(1, -1), W_out2.T, b_out2.reshape(1, -1),
      g_out.reshape(1, -1), be_out.reshape(1, -1))
    return out


# speedup vs baseline: 1.7983x; 1.7983x over previous
import math

import jax
import jax.numpy as jnp
from jax.experimental import pallas as pl
from jax.experimental.pallas import tpu as pltpu

N = 2048
I = 16
H = 16
O = 16
CUTS = 8
RADIUS = 9.0
SCALE = 1.0 / (math.sqrt(2.0) * RADIUS)
LOG2E = 1.4426950408889634
SQL2E = math.sqrt(LOG2E)

S = 32
NBLK = N // S
JC = 512
NCH = N // JC


def _lrelu(x):
    return jnp.where(x >= 0, x, 0.2 * x)


def _group_norm(x, gamma, beta, groups=4, eps=1e-5):
    c = x.shape[1]
    gs = c // groups
    cols = []
    for g in range(groups):
        sub = x[:, g * gs:(g + 1) * gs]
        m = jnp.mean(sub)
        v = jnp.mean((sub - m) * (sub - m))
        cols.append((sub - m) / jnp.sqrt(v + eps))
    return jnp.concatenate(cols, axis=1) * gamma + beta


def _pre_body(feat_ref, w1_ref, b1_ref, w2_ref, b2_ref, g_ref, be_ref, out_ref):
    x = _lrelu(jnp.dot(feat_ref[...], w1_ref[...],
                       preferred_element_type=jnp.float32) + b1_ref[...])
    x = _lrelu(jnp.dot(x, w2_ref[...],
                       preferred_element_type=jnp.float32) + b2_ref[...])
    out_ref[...] = _group_norm(x, g_ref[...], be_ref[...])


def _post_body(agg_ref, w1_ref, b1_ref, w2_ref, b2_ref, g_ref, be_ref, out_ref):
    x = _lrelu(jnp.dot(agg_ref[...], w1_ref[...],
                       preferred_element_type=jnp.float32) + b1_ref[...])
    x = _lrelu(jnp.dot(x, w2_ref[...],
                       preferred_element_type=jnp.float32) + b2_ref[...])
    out_ref[...] = _group_norm(x, g_ref[...], be_ref[...])


def _main_body(r_ref, nuv9_ref, pblk_ref, m_ref, bc1_ref, wbig_ref,
               bc2_ref, ft_ref, out_ref, xst_ref):
    pb = pblk_ref[...] * SCALE
    qi = jnp.sum(pb * pb, axis=1, keepdims=True)
    nuv9 = nuv9_ref[...]
    ni = nuv9[:, 0:3]

    a24 = jnp.dot(nuv9, m_ref[...], preferred_element_type=jnp.float32)

    z3 = jnp.zeros((S, 3), jnp.float32)
    z1 = jnp.zeros((S, 1), jnp.float32)
    o1 = jnp.ones((S, 1), jnp.float32)
    wplanes = [
        jnp.concatenate([ni, z3, z1, z1], axis=1),
        jnp.concatenate([z3, (2.0 * LOG2E) * pb, -LOG2E * o1, -LOG2E * qi],
                        axis=1),
    ]
    xplanes = []
    for c in range(CUTS):
        ac = jnp.concatenate(
            [a24[:, c:c + 1], a24[:, 8 + c:9 + c], a24[:, 16 + c:17 + c]],
            axis=1)
        cc = bc1_ref[0, c] - jnp.sum(ac * pb, axis=1, keepdims=True)
        xplanes.append(jnp.concatenate([z3, ac, z1, cc], axis=1))
    lwin = jnp.concatenate(wplanes, axis=0)
    lx = jnp.concatenate(xplanes, axis=0)

    uw = jnp.dot(lwin, r_ref[...], preferred_element_type=jnp.float32,
                 precision=jax.lax.Precision.HIGHEST)
    ux = jnp.dot(lx, r_ref[...], preferred_element_type=jnp.float32)
    dot = uw[0:S, :]
    negd = uw[S:2 * S, :]
    t = 2.0 - dot
    window = jnp.exp2(negd * t * t)

    for c in range(CUTS):
        xst_ref[c * S:(c + 1) * S, :] = jnp.maximum(
            ux[c * S:(c + 1) * S, :], 0.0).astype(jnp.bfloat16)

    xst = xst_ref[...]
    wbig = wbig_ref[...]
    y0 = jnp.dot(wbig[:H * S // 2, :], xst,
                 preferred_element_type=jnp.float32)
    y1 = jnp.dot(wbig[H * S // 2:, :], xst,
                 preferred_element_type=jnp.float32)
    cols = []
    for h in range(H):
        yh = y0 if h < 8 else y1
        hh = h if h < 8 else h - 8
        xh = jnp.maximum(yh[hh * S:(hh + 1) * S, :] + bc2_ref[0, h], 0.0)
        th = window * xh * ft_ref[h:h + 1, :]
        cols.append(jnp.sum(th, axis=1, keepdims=True))
    out_ref[...] = jnp.concatenate(cols, axis=1)


def kernel(points, nuv, features, W_in1, b_in1, W_in2, b_in2, g_in, be_in,
           Wc1, bc1, Wc2, bc2, W_out1, b_out1, W_out2, b_out2, g_out, be_out):
    nuv9 = nuv.reshape(N, 9)
    pts = points.T * SCALE
    qj = jnp.sum(pts * pts, axis=0, keepdims=True)
    nt = nuv[:, 0, :].T
    rmat = jnp.concatenate(
        [nt, pts, qj, jnp.ones((1, N), jnp.float32)], axis=0)
    m = (Wc1.T[:, None, None, :]
         * jnp.eye(3, dtype=jnp.float32)[None, :, :, None]).reshape(9, 24)
    wbig = (Wc2[:, None, :, None]
            * jnp.eye(S, dtype=jnp.float32)[None, :, None, :]).reshape(
                H * S, CUTS * S).astype(jnp.bfloat16)

    f = pl.pallas_call(
        _pre_body,
        out_shape=jax.ShapeDtypeStruct((N, H), jnp.float32),
    )(features, W_in1.T, b_in1.reshape(1, -1), W_in2.T, b_in2.reshape(1, -1),
      g_in.reshape(1, -1), be_in.reshape(1, -1))
    ft = f.T

    agg = pl.pallas_call(
        _main_body,
        grid=(NBLK,),
        in_specs=[
            pl.BlockSpec((8, N), lambda b: (0, 0)),
            pl.BlockSpec((S, 9), lambda b: (b, 0)),
            pl.BlockSpec((S, 3), lambda b: (b, 0)),
            pl.BlockSpec((9, 24), lambda b: (0, 0)),
            pl.BlockSpec((1, CUTS), lambda b: (0, 0)),
            pl.BlockSpec((H * S, CUTS * S), lambda b: (0, 0)),
            pl.BlockSpec((1, H), lambda b: (0, 0)),
            pl.BlockSpec((H, N), lambda b: (0, 0)),
        ],
        out_specs=pl.BlockSpec((S, H), lambda b: (b, 0)),
        out_shape=jax.ShapeDtypeStruct((N, H), jnp.float32),
        scratch_shapes=[pltpu.VMEM((CUTS * S, N), jnp.bfloat16)],
    )(rmat, nuv9, points, m, bc1.reshape(1, -1), wbig, bc2.reshape(1, -1),
      ft)

    out = pl.pallas_call(
        _post_body,
        out_shape=jax.ShapeDtypeStruct((N, O), jnp.float32),
    )(agg, W_out1.T, b_out1.reshape(1, -1), W_out2.T, b_out2.reshape(1, -1),
      g_out.reshape(1, -1), be_out.reshape(1, -1))
    return out
